# baseline (speedup 1.0000x reference)
import functools

import jax
import jax.numpy as jnp
from jax import lax
from jax.experimental import pallas as pl
from jax.experimental.pallas import tpu as pltpu
from jax.experimental.pallas import tpu_sc as plsc

EMBED = 128
HIDDEN = 512
VOCAB = 100000
B = 64
L = 20
T = L + 1
N_TILE = 2048


def _embed_gather(idx_flat, table):
    info = plsc.get_sparse_core_info()
    nw = info.num_cores * info.num_subcores
    n = idx_flat.shape[0]
    b_per_w = n // nw
    mesh = plsc.VectorSubcoreMesh(core_axis_name="c", subcore_axis_name="s")

    @functools.partial(
        pl.kernel,
        mesh=mesh,
        out_type=jax.ShapeDtypeStruct((n, EMBED), jnp.float32),
        scratch_types=[
            pltpu.VMEM((b_per_w,), jnp.int32),
            pltpu.VMEM((b_per_w, EMBED), jnp.float32),
            pltpu.SemaphoreType.DMA,
        ],
    )
    def gather_kernel(idx_hbm, table_hbm, out_hbm, idx_v, rows_v, sem):
        wid = lax.axis_index("s") * info.num_cores + lax.axis_index("c")
        base = wid * b_per_w
        pltpu.sync_copy(idx_hbm.at[pl.ds(base, b_per_w)], idx_v)
        pltpu.async_copy(table_hbm.at[idx_v], rows_v, sem).wait()
        pltpu.sync_copy(rows_v, out_hbm.at[pl.ds(base, b_per_w)])

    return gather_kernel(idx_flat, table)


def _lstm_body(x_ref, wih0_ref, whh0_ref, bi0_ref, bh0_ref,
               wih1_ref, whh1_ref, bi1_ref, bh1_ref, out_ref):
    b0 = bi0_ref[...] + bh0_ref[...]
    b1 = bi1_ref[...] + bh1_ref[...]
    wih0 = wih0_ref[...]
    whh0 = whh0_ref[...]
    wih1 = wih1_ref[...]
    whh1 = whh1_ref[...]

    def cell(x, w_ih, h, w_hh, b, c):
        g = (jnp.dot(x, w_ih, preferred_element_type=jnp.float32)
             + jnp.dot(h, w_hh, preferred_element_type=jnp.float32) + b)
        i = jax.nn.sigmoid(g[:, 0 * HIDDEN:1 * HIDDEN])
        f = jax.nn.sigmoid(g[:, 1 * HIDDEN:2 * HIDDEN])
        gg = jnp.tanh(g[:, 2 * HIDDEN:3 * HIDDEN])
        o = jax.nn.sigmoid(g[:, 3 * HIDDEN:4 * HIDDEN])
        c_new = f * c + i * gg
        h_new = o * jnp.tanh(c_new)
        return h_new, c_new

    def step(t, carry):
        h0, c0, h1, c1 = carry
        h0, c0 = cell(x_ref[t], wih0, h0, whh0, b0, c0)
        h1, c1 = cell(h0, wih1, h1, whh1, b1, c1)
        out_ref[t] = h1
        return (h0, c0, h1, c1)

    z = jnp.zeros((B, HIDDEN), jnp.float32)
    lax.fori_loop(0, T, step, (z, z, z, z))


def _lstm2(x_seq, wih0t, whh0t, bi0, bh0, wih1t, whh1t, bi1, bh1):
    return pl.pallas_call(
        _lstm_body,
        out_shape=jax.ShapeDtypeStruct((T, B, HIDDEN), jnp.float32),
    )(x_seq, wih0t, whh0t, bi0, bh0, wih1t, whh1t, bi1, bh1)


def _proj_body(x_ref, w_ref, b_ref, out_ref):
    out_ref[...] = lax.dot_general(
        x_ref[...], w_ref[...],
        (((1,), (1,)), ((), ())),
        preferred_element_type=jnp.float32) + b_ref[...]


def _proj(x, w, b2d):
    m = x.shape[0]
    return pl.pallas_call(
        _proj_body,
        grid=(pl.cdiv(VOCAB, N_TILE),),
        in_specs=[
            pl.BlockSpec((m, HIDDEN), lambda i: (0, 0)),
            pl.BlockSpec((N_TILE, HIDDEN), lambda i: (i, 0)),
            pl.BlockSpec((1, N_TILE), lambda i: (0, i)),
        ],
        out_specs=pl.BlockSpec((m, N_TILE), lambda i: (0, i)),
        out_shape=jax.ShapeDtypeStruct((m, VOCAB), jnp.float32),
        compiler_params=pltpu.CompilerParams(
            dimension_semantics=("arbitrary",)),
    )(x, w, b2d)


def kernel(features, captions, embedding, W_ih0, W_hh0, b_ih0, b_hh0,
           W_ih1, W_hh1, b_ih1, b_hh1, W_out, b_out):
    idx = jnp.asarray(captions, jnp.int32).reshape(-1)
    emb = _embed_gather(idx, embedding).reshape(B, L, EMBED)
    x = jnp.concatenate([features, emb], axis=1)
    x_seq = jnp.swapaxes(x, 0, 1)
    h = _lstm2(x_seq,
               W_ih0.T, W_hh0.T,
               b_ih0.reshape(1, -1), b_hh0.reshape(1, -1),
               W_ih1.T, W_hh1.T,
               b_ih1.reshape(1, -1), b_hh1.reshape(1, -1))
    hx = jnp.swapaxes(h, 0, 1).reshape(B * T, HIDDEN)
    out = _proj(hx, W_out, b_out.reshape(1, VOCAB))
    return out.reshape(B, T, VOCAB)
